# Initial kernel scaffold; baseline (speedup 1.0000x reference)
import jax
import jax.numpy as jnp
from jax.experimental import pallas as pl


def kernel(predictions, targets):
    raise NotImplementedError("write your pallas kernel here")


# speedup vs baseline: 2.4010x; 2.4010x over previous
import functools

import jax
import jax.numpy as jnp
from jax import lax
from jax.experimental import pallas as pl
from jax.experimental.pallas import tpu as pltpu
from jax.experimental.pallas import tpu_sc as plsc

NUM_BINS = 10
VARIANCE_WEIGHT = 0.1
RANGE_WEIGHT = 0.05

_NC = 2
_NS = 16
_L = 16
_NW = _NC * _NS
_CHUNK = 16384
_UNROLL = 8
_SEC = 512

_EDGES = (
    0.10000000149011612,
    0.20000000298023224,
    0.30000001192092896,
    0.4000000059604645,
    0.5,
    0.6000000238418579,
    0.699999988079071,
    0.800000011920929,
    0.9000000357627869,
)


def _sc_body(p_hbm, t_hbm, part_hbm,
             pb0, pb1, tb0, tb1, stage_v, sem0, sem1):
    wid = lax.axis_index("s") * _NC + lax.axis_index("c")
    n = p_hbm.shape[0]
    per_w = n // _NW
    n_chunks = per_w // _CHUNK
    base = wid * per_w

    zeros16 = jnp.zeros((_L,), jnp.float32)
    inf16 = jnp.full((_L,), jnp.inf, jnp.float32)

    pbufs = (pb0, pb1)
    tbufs = (tb0, tb1)
    sems = (sem0, sem1)

    def start(c, b):
        off = base + c * _CHUNK
        pltpu.async_copy(p_hbm.at[pl.ds(off, _CHUNK)], pbufs[b], sems[b])
        pltpu.async_copy(t_hbm.at[pl.ds(off, _CHUNK)], tbufs[b], sems[b])

    def wait(b):
        pltpu.make_async_copy(
            p_hbm.at[pl.ds(0, _CHUNK)], pbufs[b], sems[b]).wait()
        pltpu.make_async_copy(
            t_hbm.at[pl.ds(0, _CHUNK)], tbufs[b], sems[b]).wait()

    start(0, 0)

    def chunk_compute(pb, tb, carry):
        def body(v, carry):
            (cnt, tj, sqtot, sp, sp2, st, st2, mnp, mxp, mnt, mxt) = carry
            cnt = list(cnt)
            tj = list(tj)
            for u in range(_UNROLL):
                off = v * (_L * _UNROLL) + u * _L
                p16 = pb[pl.ds(off, _L)]
                t16 = tb[pl.ds(off, _L)]
                d = p16 - t16
                sqv = d * d
                for j in range(9):
                    m = t16 > _EDGES[j]
                    cnt[j] = cnt[j] + jnp.where(m, 1.0, 0.0)
                    tj[j] = tj[j] + jnp.where(m, sqv, 0.0)
                sqtot = sqtot + sqv
                sp = sp + p16
                st = st + t16
                sp2 = sp2 + p16 * p16
                st2 = st2 + t16 * t16
                mnp = jnp.minimum(mnp, p16)
                mxp = jnp.maximum(mxp, p16)
                mnt = jnp.minimum(mnt, t16)
                mxt = jnp.maximum(mxt, t16)
            return (tuple(cnt), tuple(tj), sqtot,
                    sp, sp2, st, st2, mnp, mxp, mnt, mxt)
        return lax.fori_loop(0, _CHUNK // (_L * _UNROLL), body, carry)

    carry = ((zeros16,) * 9, (zeros16,) * 9, zeros16,
             zeros16, zeros16, zeros16, zeros16,
             inf16, -inf16, inf16, -inf16)

    def outer(g, carry):
        for b in range(2):
            c = g * 2 + b

            @pl.when(c + 1 < n_chunks)
            def _():
                start(c + 1, b ^ 1)

            wait(b)
            carry = chunk_compute(pbufs[b], tbufs[b], carry)
        return carry

    carry = lax.fori_loop(0, n_chunks // 2, outer, carry)
    (cnt, tj, sqtot, sp, sp2, st, st2, mnp, mxp, mnt, mxt) = carry

    vecs = (list(cnt) + list(tj)
            + [sqtot, sp, sp2, st, st2, mnp, mxp, mnt, mxt])
    for i, vec in enumerate(vecs):
        stage_v[pl.ds(i * _L, _L)] = vec
    pltpu.sync_copy(stage_v, part_hbm.at[wid])


def _combine_kernel(part_ref, out_ref, *, n_total):
    x = part_ref[...]
    n_f = jnp.float32(n_total)

    def sec(i):
        return x[:, i * _L:(i + 1) * _L]

    G = [jnp.sum(sec(j)) for j in range(9)]
    T = [jnp.sum(sec(9 + j)) for j in range(9)]
    sqtot = jnp.sum(sec(18))
    sum_p = jnp.sum(sec(19))
    sum_p2 = jnp.sum(sec(20))
    sum_t = jnp.sum(sec(21))
    sum_t2 = jnp.sum(sec(22))
    min_p = jnp.min(sec(23))
    max_p = jnp.max(sec(24))
    min_t = jnp.min(sec(25))
    max_t = jnp.max(sec(26))

    cnts = [n_f - G[0]] + [G[b - 1] - G[b] for b in range(1, 9)] + [G[8]]
    S = [sqtot - T[0]] + [T[b - 1] - T[b] for b in range(1, 9)] + [T[8]]

    w_raw = [jnp.where(c > 0.0, 1.0 / (c + 1e-6), jnp.float32(0.0))
             for c in cnts]
    w_sum = functools.reduce(lambda a, b: a + b, w_raw)
    w_mean = w_sum / jnp.float32(NUM_BINS)
    wS = functools.reduce(lambda a, b: a + b,
                          [w * s for w, s in zip(w_raw, S)])
    mse = wS / w_mean / n_f

    var_p = sum_p2 / n_f - (sum_p / n_f) ** 2
    var_t = sum_t2 / n_f - (sum_t / n_f) ** 2
    variance_loss = jnp.abs(var_p - var_t)
    range_loss = jnp.maximum(
        jnp.float32(0.0), (max_t - min_t) * 0.5 - (max_p - min_p))
    out_ref[0, 0] = (mse + VARIANCE_WEIGHT * variance_loss
                     + RANGE_WEIGHT * range_loss)


def kernel(predictions, targets):
    n = predictions.shape[0]

    mesh = plsc.VectorSubcoreMesh(
        core_axis_name="c", subcore_axis_name="s",
        num_cores=_NC, num_subcores=_NS)
    parts = pl.kernel(
        _sc_body,
        out_type=jax.ShapeDtypeStruct((_NW, _SEC), jnp.float32),
        mesh=mesh,
        scratch_types=[
            pltpu.VMEM((_CHUNK,), jnp.float32),
            pltpu.VMEM((_CHUNK,), jnp.float32),
            pltpu.VMEM((_CHUNK,), jnp.float32),
            pltpu.VMEM((_CHUNK,), jnp.float32),
            pltpu.VMEM((_SEC,), jnp.float32),
            pltpu.SemaphoreType.DMA,
            pltpu.SemaphoreType.DMA,
        ],
    )(predictions, targets)

    out = pl.pallas_call(
        functools.partial(_combine_kernel, n_total=n),
        in_specs=[pl.BlockSpec((_NW, _SEC), lambda: (0, 0))],
        out_specs=pl.BlockSpec(memory_space=pltpu.SMEM),
        out_shape=jax.ShapeDtypeStruct((1, 1), jnp.float32),
        interpret=False,
    )(parts)
    return out.reshape(())


# speedup vs baseline: 35.8825x; 14.9450x over previous
import functools

import jax
import jax.numpy as jnp
from jax import lax
from jax.experimental import pallas as pl
from jax.experimental.pallas import tpu as pltpu
from jax.experimental.pallas import tpu_sc as plsc

NUM_BINS = 10
VARIANCE_WEIGHT = 0.1
RANGE_WEIGHT = 0.05

_NC = 2
_NS = 16
_L = 16
_NW = _NC * _NS
_CHUNK = 16384
_UNROLL = 2
_SEC = 512

_SC_SHARE_UNITS = 8

_BLK_ROWS = 1024
_LANES = 128
_SUB = 8

_EDGES = (
    0.10000000149011612,
    0.20000000298023224,
    0.30000001192092896,
    0.4000000059604645,
    0.5,
    0.6000000238418579,
    0.699999988079071,
    0.800000011920929,
    0.9000000357627869,
)


def _sc_body(p_hbm, t_hbm, part_hbm,
             pb0, pb1, tb0, tb1, stage_v, sem0, sem1, *, sc_base, sc_n):
    wid = lax.axis_index("s") * _NC + lax.axis_index("c")
    per_w = sc_n // _NW
    n_chunks = per_w // _CHUNK
    base = sc_base + wid * per_w

    zeros16 = jnp.zeros((_L,), jnp.float32)
    inf16 = jnp.full((_L,), jnp.inf, jnp.float32)

    pbufs = (pb0, pb1)
    tbufs = (tb0, tb1)
    sems = (sem0, sem1)

    def start(c, b):
        off = base + c * _CHUNK
        pltpu.async_copy(p_hbm.at[pl.ds(off, _CHUNK)], pbufs[b], sems[b])
        pltpu.async_copy(t_hbm.at[pl.ds(off, _CHUNK)], tbufs[b], sems[b])

    def wait(b):
        pltpu.make_async_copy(
            p_hbm.at[pl.ds(0, _CHUNK)], pbufs[b], sems[b]).wait()
        pltpu.make_async_copy(
            t_hbm.at[pl.ds(0, _CHUNK)], tbufs[b], sems[b]).wait()

    start(0, 0)

    def chunk_compute(pb, tb, carry):
        def body(v, carry):
            (cnt, tj, sqtot, sp, sp2, st, st2, mnp, mxp, mnt, mxt) = carry
            cnt = list(cnt)
            tj = list(tj)
            for u in range(_UNROLL):
                off = v * (_L * _UNROLL) + u * _L
                p16 = pb[pl.ds(off, _L)]
                t16 = tb[pl.ds(off, _L)]
                d = p16 - t16
                sqv = d * d
                for j in range(9):
                    m = t16 > _EDGES[j]
                    cnt[j] = cnt[j] + jnp.where(m, 1.0, 0.0)
                    tj[j] = tj[j] + jnp.where(m, sqv, 0.0)
                sqtot = sqtot + sqv
                sp = sp + p16
                st = st + t16
                sp2 = sp2 + p16 * p16
                st2 = st2 + t16 * t16
                mnp = jnp.minimum(mnp, p16)
                mxp = jnp.maximum(mxp, p16)
                mnt = jnp.minimum(mnt, t16)
                mxt = jnp.maximum(mxt, t16)
            return (tuple(cnt), tuple(tj), sqtot,
                    sp, sp2, st, st2, mnp, mxp, mnt, mxt)
        return lax.fori_loop(0, _CHUNK // (_L * _UNROLL), body, carry)

    carry = ((zeros16,) * 9, (zeros16,) * 9, zeros16,
             zeros16, zeros16, zeros16, zeros16,
             inf16, -inf16, inf16, -inf16)

    def outer(g, carry):
        for b in range(2):
            c = g * 2 + b

            @pl.when(c + 1 < n_chunks)
            def _():
                start(c + 1, b ^ 1)

            wait(b)
            carry = chunk_compute(pbufs[b], tbufs[b], carry)
        return carry

    carry = lax.fori_loop(0, n_chunks // 2, outer, carry)
    (cnt, tj, sqtot, sp, sp2, st, st2, mnp, mxp, mnt, mxt) = carry

    vecs = (list(cnt) + list(tj)
            + [sqtot, sp, sp2, st, st2, mnp, mxp, mnt, mxt])
    for i, vec in enumerate(vecs):
        stage_v[pl.ds(i * _L, _L)] = vec
    pltpu.sync_copy(stage_v, part_hbm.at[wid])


def _tc_partial_kernel(p_ref, t_ref, out_ref, acc_ref, mm_ref, *, nsteps):
    pid = pl.program_id(0)

    @pl.when(pid == 0)
    def _init():
        acc_ref[...] = jnp.zeros_like(acc_ref)
        inf = jnp.full((_SUB, _LANES), jnp.inf, dtype=jnp.float32)
        mm_ref[0] = inf
        mm_ref[1] = -inf
        mm_ref[2] = inf
        mm_ref[3] = -inf

    p = p_ref[...]
    t = t_ref[...]
    d = p - t
    sq = d * d

    def rsum(x):
        return jnp.sum(x.reshape(_BLK_ROWS // _SUB, _SUB, _LANES), axis=0)

    for j in range(9):
        m = (t > _EDGES[j]).astype(jnp.float32)
        acc_ref[j] = acc_ref[j] + rsum(m)
        acc_ref[9 + j] = acc_ref[9 + j] + rsum(m * sq)

    acc_ref[18] = acc_ref[18] + rsum(sq)
    acc_ref[19] = acc_ref[19] + rsum(p)
    acc_ref[20] = acc_ref[20] + rsum(p * p)
    acc_ref[21] = acc_ref[21] + rsum(t)
    acc_ref[22] = acc_ref[22] + rsum(t * t)

    p3 = p.reshape(_BLK_ROWS // _SUB, _SUB, _LANES)
    t3 = t.reshape(_BLK_ROWS // _SUB, _SUB, _LANES)
    mm_ref[0] = jnp.minimum(mm_ref[0], jnp.min(p3, axis=0))
    mm_ref[1] = jnp.maximum(mm_ref[1], jnp.max(p3, axis=0))
    mm_ref[2] = jnp.minimum(mm_ref[2], jnp.min(t3, axis=0))
    mm_ref[3] = jnp.maximum(mm_ref[3], jnp.max(t3, axis=0))

    @pl.when(pid == nsteps - 1)
    def _epilogue():
        for j in range(23):
            out_ref[0, j] = jnp.sum(acc_ref[j])
        out_ref[0, 23] = jnp.min(mm_ref[0])
        out_ref[0, 24] = jnp.max(mm_ref[1])
        out_ref[0, 25] = jnp.min(mm_ref[2])
        out_ref[0, 26] = jnp.max(mm_ref[3])


def _combine_kernel(sc_ref, tc_ref, out_ref, *, n_total):
    x = sc_ref[...]
    n_f = jnp.float32(n_total)

    def sec(i):
        return x[:, i * _L:(i + 1) * _L]

    def tc(i):
        return tc_ref[0, i]

    G = [jnp.sum(sec(j)) + tc(j) for j in range(9)]
    T = [jnp.sum(sec(9 + j)) + tc(9 + j) for j in range(9)]
    sqtot = jnp.sum(sec(18)) + tc(18)
    sum_p = jnp.sum(sec(19)) + tc(19)
    sum_p2 = jnp.sum(sec(20)) + tc(20)
    sum_t = jnp.sum(sec(21)) + tc(21)
    sum_t2 = jnp.sum(sec(22)) + tc(22)
    min_p = jnp.minimum(jnp.min(sec(23)), tc(23))
    max_p = jnp.maximum(jnp.max(sec(24)), tc(24))
    min_t = jnp.minimum(jnp.min(sec(25)), tc(25))
    max_t = jnp.maximum(jnp.max(sec(26)), tc(26))

    cnts = [n_f - G[0]] + [G[b - 1] - G[b] for b in range(1, 9)] + [G[8]]
    S = [sqtot - T[0]] + [T[b - 1] - T[b] for b in range(1, 9)] + [T[8]]

    w_raw = [jnp.where(c > 0.0, 1.0 / (c + 1e-6), jnp.float32(0.0))
             for c in cnts]
    w_sum = functools.reduce(lambda a, b: a + b, w_raw)
    w_mean = w_sum / jnp.float32(NUM_BINS)
    wS = functools.reduce(lambda a, b: a + b,
                          [w * s for w, s in zip(w_raw, S)])
    mse = wS / w_mean / n_f

    var_p = sum_p2 / n_f - (sum_p / n_f) ** 2
    var_t = sum_t2 / n_f - (sum_t / n_f) ** 2
    variance_loss = jnp.abs(var_p - var_t)
    range_loss = jnp.maximum(
        jnp.float32(0.0), (max_t - min_t) * 0.5 - (max_p - min_p))
    out_ref[0, 0] = (mse + VARIANCE_WEIGHT * variance_loss
                     + RANGE_WEIGHT * range_loss)


def kernel(predictions, targets):
    n = predictions.shape[0]
    sc_n = _SC_SHARE_UNITS * _NW * _CHUNK
    tc_n = n - sc_n
    nsteps = tc_n // (_BLK_ROWS * _LANES)
    rows = n // _LANES

    mesh = plsc.VectorSubcoreMesh(
        core_axis_name="c", subcore_axis_name="s",
        num_cores=_NC, num_subcores=_NS)
    sc_parts = pl.kernel(
        functools.partial(_sc_body, sc_base=tc_n, sc_n=sc_n),
        out_type=jax.ShapeDtypeStruct((_NW, _SEC), jnp.float32),
        mesh=mesh,
        scratch_types=[
            pltpu.VMEM((_CHUNK,), jnp.float32),
            pltpu.VMEM((_CHUNK,), jnp.float32),
            pltpu.VMEM((_CHUNK,), jnp.float32),
            pltpu.VMEM((_CHUNK,), jnp.float32),
            pltpu.VMEM((_SEC,), jnp.float32),
            pltpu.SemaphoreType.DMA,
            pltpu.SemaphoreType.DMA,
        ],
    )(predictions, targets)

    p2 = predictions.reshape(rows, _LANES)
    t2 = targets.reshape(rows, _LANES)
    tc_parts = pl.pallas_call(
        functools.partial(_tc_partial_kernel, nsteps=nsteps),
        grid=(nsteps,),
        in_specs=[
            pl.BlockSpec((_BLK_ROWS, _LANES), lambda i: (i, 0)),
            pl.BlockSpec((_BLK_ROWS, _LANES), lambda i: (i, 0)),
        ],
        out_specs=pl.BlockSpec(memory_space=pltpu.SMEM),
        out_shape=jax.ShapeDtypeStruct((1, 27), jnp.float32),
        scratch_shapes=[
            pltpu.VMEM((23, _SUB, _LANES), jnp.float32),
            pltpu.VMEM((4, _SUB, _LANES), jnp.float32),
        ],
        interpret=False,
    )(p2, t2)

    out = pl.pallas_call(
        functools.partial(_combine_kernel, n_total=n),
        in_specs=[
            pl.BlockSpec((_NW, _SEC), lambda: (0, 0)),
            pl.BlockSpec(memory_space=pltpu.SMEM),
        ],
        out_specs=pl.BlockSpec(memory_space=pltpu.SMEM),
        out_shape=jax.ShapeDtypeStruct((1, 1), jnp.float32),
        interpret=False,
    )(sc_parts, tc_parts)
    return out.reshape(())


# speedup vs baseline: 35.9215x; 1.0011x over previous
import functools

import jax
import jax.numpy as jnp
from jax import lax
from jax.experimental import pallas as pl
from jax.experimental.pallas import tpu as pltpu
from jax.experimental.pallas import tpu_sc as plsc

NUM_BINS = 10
VARIANCE_WEIGHT = 0.1
RANGE_WEIGHT = 0.05

_NC = 2
_NS = 16
_L = 16
_NW = _NC * _NS
_CHUNK = 16384
_UNROLL = 2
_SEC = 512

_SC_SHARE_UNITS = 8

_BLK_ROWS = 1024
_LANES = 128
_SUB = 8

_EDGES = (
    0.10000000149011612,
    0.20000000298023224,
    0.30000001192092896,
    0.4000000059604645,
    0.5,
    0.6000000238418579,
    0.699999988079071,
    0.800000011920929,
    0.9000000357627869,
)


def _sc_body(p_hbm, t_hbm, part_hbm,
             pb0, pb1, tb0, tb1, stage_v, sem0, sem1, *, sc_base, sc_n):
    wid = lax.axis_index("s") * _NC + lax.axis_index("c")
    per_w = sc_n // _NW
    n_chunks = per_w // _CHUNK
    base = sc_base + wid * per_w

    zeros16 = jnp.zeros((_L,), jnp.float32)
    inf16 = jnp.full((_L,), jnp.inf, jnp.float32)

    pbufs = (pb0, pb1)
    tbufs = (tb0, tb1)
    sems = (sem0, sem1)

    def start(c, b):
        off = base + c * _CHUNK
        pltpu.async_copy(p_hbm.at[pl.ds(off, _CHUNK)], pbufs[b], sems[b])
        pltpu.async_copy(t_hbm.at[pl.ds(off, _CHUNK)], tbufs[b], sems[b])

    def wait(b):
        pltpu.make_async_copy(
            p_hbm.at[pl.ds(0, _CHUNK)], pbufs[b], sems[b]).wait()
        pltpu.make_async_copy(
            t_hbm.at[pl.ds(0, _CHUNK)], tbufs[b], sems[b]).wait()

    start(0, 0)

    def chunk_compute(pb, tb, carry):
        def body(v, carry):
            (cnt, tj, sqtot, sp, sp2, st, st2, mnp, mxp, mnt, mxt) = carry
            cnt = list(cnt)
            tj = list(tj)
            for u in range(_UNROLL):
                off = v + u * _L
                p16 = pb[pl.ds(off, _L)]
                t16 = tb[pl.ds(off, _L)]
                d = p16 - t16
                sqv = d * d
                for j in range(9):
                    m = t16 > _EDGES[j]
                    cnt[j] = cnt[j] + jnp.where(m, 1.0, 0.0)
                    tj[j] = tj[j] + jnp.where(m, sqv, 0.0)
                sqtot = sqtot + sqv
                sp = sp + p16
                st = st + t16
                sp2 = sp2 + p16 * p16
                st2 = st2 + t16 * t16
                mnp = jnp.minimum(mnp, p16)
                mxp = jnp.maximum(mxp, p16)
                mnt = jnp.minimum(mnt, t16)
                mxt = jnp.maximum(mxt, t16)
            return (tuple(cnt), tuple(tj), sqtot,
                    sp, sp2, st, st2, mnp, mxp, mnt, mxt)
        return plsc.parallel_loop(
            0, _CHUNK, step=_L * _UNROLL, unroll=1, carry=carry)(body)

    carry = ((zeros16,) * 9, (zeros16,) * 9, zeros16,
             zeros16, zeros16, zeros16, zeros16,
             inf16, -inf16, inf16, -inf16)

    def outer(g, carry):
        for b in range(2):
            c = g * 2 + b

            @pl.when(c + 1 < n_chunks)
            def _():
                start(c + 1, b ^ 1)

            wait(b)
            carry = chunk_compute(pbufs[b], tbufs[b], carry)
        return carry

    carry = lax.fori_loop(0, n_chunks // 2, outer, carry)
    (cnt, tj, sqtot, sp, sp2, st, st2, mnp, mxp, mnt, mxt) = carry

    vecs = (list(cnt) + list(tj)
            + [sqtot, sp, sp2, st, st2, mnp, mxp, mnt, mxt])
    for i, vec in enumerate(vecs):
        stage_v[pl.ds(i * _L, _L)] = vec
    pltpu.sync_copy(stage_v, part_hbm.at[wid])


def _tc_partial_kernel(p_ref, t_ref, out_ref, acc_ref, mm_ref, *, nsteps):
    pid = pl.program_id(0)

    @pl.when(pid == 0)
    def _init():
        acc_ref[...] = jnp.zeros_like(acc_ref)
        inf = jnp.full((_SUB, _LANES), jnp.inf, dtype=jnp.float32)
        mm_ref[0] = inf
        mm_ref[1] = -inf
        mm_ref[2] = inf
        mm_ref[3] = -inf

    p = p_ref[...]
    t = t_ref[...]
    d = p - t
    sq = d * d

    def rsum(x):
        return jnp.sum(x.reshape(_BLK_ROWS // _SUB, _SUB, _LANES), axis=0)

    for j in range(9):
        m = (t > _EDGES[j]).astype(jnp.float32)
        acc_ref[j] = acc_ref[j] + rsum(m)
        acc_ref[9 + j] = acc_ref[9 + j] + rsum(m * sq)

    acc_ref[18] = acc_ref[18] + rsum(sq)
    acc_ref[19] = acc_ref[19] + rsum(p)
    acc_ref[20] = acc_ref[20] + rsum(p * p)
    acc_ref[21] = acc_ref[21] + rsum(t)
    acc_ref[22] = acc_ref[22] + rsum(t * t)

    p3 = p.reshape(_BLK_ROWS // _SUB, _SUB, _LANES)
    t3 = t.reshape(_BLK_ROWS // _SUB, _SUB, _LANES)
    mm_ref[0] = jnp.minimum(mm_ref[0], jnp.min(p3, axis=0))
    mm_ref[1] = jnp.maximum(mm_ref[1], jnp.max(p3, axis=0))
    mm_ref[2] = jnp.minimum(mm_ref[2], jnp.min(t3, axis=0))
    mm_ref[3] = jnp.maximum(mm_ref[3], jnp.max(t3, axis=0))

    @pl.when(pid == nsteps - 1)
    def _epilogue():
        for j in range(23):
            out_ref[0, j] = jnp.sum(acc_ref[j])
        out_ref[0, 23] = jnp.min(mm_ref[0])
        out_ref[0, 24] = jnp.max(mm_ref[1])
        out_ref[0, 25] = jnp.min(mm_ref[2])
        out_ref[0, 26] = jnp.max(mm_ref[3])


def _combine_kernel(sc_ref, tc_ref, out_ref, *, n_total):
    x = sc_ref[...]
    n_f = jnp.float32(n_total)

    def sec(i):
        return x[:, i * _L:(i + 1) * _L]

    def tc(i):
        return tc_ref[0, i]

    G = [jnp.sum(sec(j)) + tc(j) for j in range(9)]
    T = [jnp.sum(sec(9 + j)) + tc(9 + j) for j in range(9)]
    sqtot = jnp.sum(sec(18)) + tc(18)
    sum_p = jnp.sum(sec(19)) + tc(19)
    sum_p2 = jnp.sum(sec(20)) + tc(20)
    sum_t = jnp.sum(sec(21)) + tc(21)
    sum_t2 = jnp.sum(sec(22)) + tc(22)
    min_p = jnp.minimum(jnp.min(sec(23)), tc(23))
    max_p = jnp.maximum(jnp.max(sec(24)), tc(24))
    min_t = jnp.minimum(jnp.min(sec(25)), tc(25))
    max_t = jnp.maximum(jnp.max(sec(26)), tc(26))

    cnts = [n_f - G[0]] + [G[b - 1] - G[b] for b in range(1, 9)] + [G[8]]
    S = [sqtot - T[0]] + [T[b - 1] - T[b] for b in range(1, 9)] + [T[8]]

    w_raw = [jnp.where(c > 0.0, 1.0 / (c + 1e-6), jnp.float32(0.0))
             for c in cnts]
    w_sum = functools.reduce(lambda a, b: a + b, w_raw)
    w_mean = w_sum / jnp.float32(NUM_BINS)
    wS = functools.reduce(lambda a, b: a + b,
                          [w * s for w, s in zip(w_raw, S)])
    mse = wS / w_mean / n_f

    var_p = sum_p2 / n_f - (sum_p / n_f) ** 2
    var_t = sum_t2 / n_f - (sum_t / n_f) ** 2
    variance_loss = jnp.abs(var_p - var_t)
    range_loss = jnp.maximum(
        jnp.float32(0.0), (max_t - min_t) * 0.5 - (max_p - min_p))
    out_ref[0, 0] = (mse + VARIANCE_WEIGHT * variance_loss
                     + RANGE_WEIGHT * range_loss)


def kernel(predictions, targets):
    n = predictions.shape[0]
    sc_n = _SC_SHARE_UNITS * _NW * _CHUNK
    tc_n = n - sc_n
    nsteps = tc_n // (_BLK_ROWS * _LANES)
    rows = n // _LANES

    mesh = plsc.VectorSubcoreMesh(
        core_axis_name="c", subcore_axis_name="s",
        num_cores=_NC, num_subcores=_NS)
    sc_parts = pl.kernel(
        functools.partial(_sc_body, sc_base=tc_n, sc_n=sc_n),
        out_type=jax.ShapeDtypeStruct((_NW, _SEC), jnp.float32),
        mesh=mesh,
        scratch_types=[
            pltpu.VMEM((_CHUNK,), jnp.float32),
            pltpu.VMEM((_CHUNK,), jnp.float32),
            pltpu.VMEM((_CHUNK,), jnp.float32),
            pltpu.VMEM((_CHUNK,), jnp.float32),
            pltpu.VMEM((_SEC,), jnp.float32),
            pltpu.SemaphoreType.DMA,
            pltpu.SemaphoreType.DMA,
        ],
    )(predictions, targets)

    p2 = predictions.reshape(rows, _LANES)
    t2 = targets.reshape(rows, _LANES)
    tc_parts = pl.pallas_call(
        functools.partial(_tc_partial_kernel, nsteps=nsteps),
        grid=(nsteps,),
        in_specs=[
            pl.BlockSpec((_BLK_ROWS, _LANES), lambda i: (i, 0)),
            pl.BlockSpec((_BLK_ROWS, _LANES), lambda i: (i, 0)),
        ],
        out_specs=pl.BlockSpec(memory_space=pltpu.SMEM),
        out_shape=jax.ShapeDtypeStruct((1, 27), jnp.float32),
        scratch_shapes=[
            pltpu.VMEM((23, _SUB, _LANES), jnp.float32),
            pltpu.VMEM((4, _SUB, _LANES), jnp.float32),
        ],
        interpret=False,
    )(p2, t2)

    out = pl.pallas_call(
        functools.partial(_combine_kernel, n_total=n),
        in_specs=[
            pl.BlockSpec((_NW, _SEC), lambda: (0, 0)),
            pl.BlockSpec(memory_space=pltpu.SMEM),
        ],
        out_specs=pl.BlockSpec(memory_space=pltpu.SMEM),
        out_shape=jax.ShapeDtypeStruct((1, 1), jnp.float32),
        interpret=False,
    )(sc_parts, tc_parts)
    return out.reshape(())
